# Initial kernel scaffold; baseline (speedup 1.0000x reference)
import jax
import jax.numpy as jnp
from jax.experimental import pallas as pl


def kernel(x, edge_index, edge_weight, W_ih, W_hh, b_ih, b_hh, fc_W, fc_b):
    raise NotImplementedError("write your pallas kernel here")


# speedup vs baseline: 3.5090x; 3.5090x over previous
import functools

import jax
import jax.numpy as jnp
from jax import lax
from jax.experimental import pallas as pl
from jax.experimental.pallas import tpu as pltpu
from jax.experimental.pallas import tpu_sc as plsc

NHID = 128
NGATE = 4 * NHID

_NC = 2
_NS = 16
_L = 16
_NW = _NC * _NS
_CHUNK = 128


def _splat(wv, k):
    dnums = lax.GatherDimensionNumbers(
        offset_dims=(), collapsed_slice_dims=(0,), start_index_map=(0,))
    idx = jnp.full((_L, 1), k, dtype=jnp.int32)
    return lax.gather(wv, idx, dnums, slice_sizes=(1,),
                      mode=lax.GatherScatterMode.PROMISE_IN_BOUNDS)


def _make_spmm(n, e_pad, d):
    cpw = e_pad // (_NW * _CHUNK)
    stripe = (n // _NS) // 8 * 8
    tail = n - (_NS - 1) * stripe
    mesh = plsc.VectorSubcoreMesh(core_axis_name="c", subcore_axis_name="s")

    @functools.partial(
        pl.kernel,
        out_type=jax.ShapeDtypeStruct((_NC, n, d), jnp.float32),
        mesh=mesh,
        scratch_types=[
            pltpu.VMEM((_CHUNK,), jnp.int32),
            pltpu.VMEM((_CHUNK,), jnp.int32),
            pltpu.VMEM((_CHUNK,), jnp.float32),
            pltpu.VMEM((_CHUNK, d), jnp.float32),
            pltpu.VMEM_SHARED((n, d), jnp.float32),
            pltpu.SemaphoreType.DMA,
        ],
    )
    def spmm(h_hbm, src_hbm, dst_hbm, w_hbm, out_hbm,
             src_v, dst_v, w_v, rows_v, acc, sem):
        cid = lax.axis_index("c")
        sid = lax.axis_index("s")
        wid = sid * _NC + cid

        zeros = jnp.zeros((_L,), jnp.float32)

        def zrow(i, _):
            for j in range(d // _L):
                rows_v[i, pl.ds(j * _L, _L)] = zeros
            return 0

        lax.fori_loop(0, _CHUNK, zrow, 0)
        r0 = sid * stripe
        nfull = stripe // _CHUNK
        rem = stripe - nfull * _CHUNK

        def crow(b, _):
            pltpu.sync_copy(rows_v, acc.at[pl.ds(r0 + b * _CHUNK, _CHUNK)])
            return 0

        lax.fori_loop(0, nfull, crow, 0)
        if rem:
            pltpu.sync_copy(rows_v.at[pl.ds(0, rem)],
                            acc.at[pl.ds(r0 + nfull * _CHUNK, rem)])

        @pl.when(sid == _NS - 1)
        def _():
            pltpu.sync_copy(rows_v.at[pl.ds(0, tail - stripe)],
                            acc.at[pl.ds(r0 + stripe, tail - stripe)])

        plsc.subcore_barrier()

        base0 = wid * (cpw * _CHUNK)

        def chunk_body(g, _):
            base = base0 + g * _CHUNK
            pltpu.sync_copy(src_hbm.at[pl.ds(base, _CHUNK)], src_v)
            pltpu.sync_copy(dst_hbm.at[pl.ds(base, _CHUNK)], dst_v)
            pltpu.sync_copy(w_hbm.at[pl.ds(base, _CHUNK)], w_v)
            pltpu.async_copy(h_hbm.at[src_v], rows_v, sem).wait()

            def group_body(g2, _):
                wv = w_v[pl.ds(g2 * _L, _L)]
                for k in range(_L):
                    ws = _splat(wv, k)
                    ei = g2 * _L + k
                    for j in range(d // _L):
                        sl = pl.ds(j * _L, _L)
                        rows_v[ei, sl] = rows_v[ei, sl] * ws
                return 0

            lax.fori_loop(0, _CHUNK // _L, group_body, 0)
            pltpu.sync_copy(rows_v, acc.at[dst_v], add=True)
            return 0

        lax.fori_loop(0, cpw, chunk_body, 0)
        plsc.subcore_barrier()
        pltpu.sync_copy(acc.at[pl.ds(r0, stripe)],
                        out_hbm.at[cid, pl.ds(r0, stripe)])

        @pl.when(sid == _NS - 1)
        def _():
            pltpu.sync_copy(acc.at[pl.ds(r0 + stripe, tail - stripe)],
                            out_hbm.at[cid, pl.ds(r0 + stripe, tail - stripe)])

    return spmm


def _k1_body(x_ref, wT_ref, bi_ref, bh_ref, xw_ref, h_ref, c_ref):
    xw = jnp.dot(x_ref[...], wT_ref[...], preferred_element_type=jnp.float32)
    xw = xw + bi_ref[...] + bh_ref[...]
    xw_ref[...] = xw
    i = jax.nn.sigmoid(xw[:, 0 * NHID:1 * NHID])
    g = jnp.tanh(xw[:, 2 * NHID:3 * NHID])
    o = jax.nn.sigmoid(xw[:, 3 * NHID:4 * NHID])
    c = i * g
    c_ref[...] = c
    h_ref[...] = o * jnp.tanh(c)


def _cell_body(xw_ref, p_ref, c_ref, whT_ref, h_ref, cn_ref):
    h_in = p_ref[0] + p_ref[1]
    gates = xw_ref[...] + jnp.dot(h_in, whT_ref[...],
                                  preferred_element_type=jnp.float32)
    i = jax.nn.sigmoid(gates[:, 0 * NHID:1 * NHID])
    f = jax.nn.sigmoid(gates[:, 1 * NHID:2 * NHID])
    g = jnp.tanh(gates[:, 2 * NHID:3 * NHID])
    o = jax.nn.sigmoid(gates[:, 3 * NHID:4 * NHID])
    c = f * c_ref[...] + i * g
    cn_ref[...] = c
    h_ref[...] = o * jnp.tanh(c)


def _fc_body(p_ref, fwT_ref, fb_ref, o_ref):
    h = jax.nn.relu(p_ref[0] + p_ref[1])
    o_ref[...] = jnp.dot(h, fwT_ref[...],
                         preferred_element_type=jnp.float32) + fb_ref[...]


def kernel(x, edge_index, edge_weight, W_ih, W_hh, b_ih, b_hh, fc_W, fc_b):
    n, n_in = x.shape
    e = edge_index.shape[1]
    d = W_hh.shape[1]

    W_ihT = W_ih.T
    W_hhT = W_hh.T
    fc_WT = fc_W.T
    bi = b_ih.reshape(1, NGATE)
    bh = b_hh.reshape(1, NGATE)
    fb = fc_b.reshape(1, fc_W.shape[0])

    per_w = _NW * _CHUNK
    e_pad = ((e + per_w - 1) // per_w) * per_w
    pad = e_pad - e
    dst = jnp.pad(edge_index[0], (0, pad))
    src = jnp.pad(edge_index[1], (0, pad))
    w = jnp.pad(edge_weight, (0, pad))

    spmm = _make_spmm(n, e_pad, d)

    blk = 1000
    grid = n // blk

    k1 = pl.pallas_call(
        _k1_body,
        grid=(grid,),
        in_specs=[
            pl.BlockSpec((blk, n_in), lambda i: (i, 0)),
            pl.BlockSpec((n_in, NGATE), lambda i: (0, 0)),
            pl.BlockSpec((1, NGATE), lambda i: (0, 0)),
            pl.BlockSpec((1, NGATE), lambda i: (0, 0)),
        ],
        out_specs=[
            pl.BlockSpec((blk, NGATE), lambda i: (i, 0)),
            pl.BlockSpec((blk, d), lambda i: (i, 0)),
            pl.BlockSpec((blk, d), lambda i: (i, 0)),
        ],
        out_shape=[
            jax.ShapeDtypeStruct((n, NGATE), jnp.float32),
            jax.ShapeDtypeStruct((n, d), jnp.float32),
            jax.ShapeDtypeStruct((n, d), jnp.float32),
        ],
    )

    cell = pl.pallas_call(
        _cell_body,
        grid=(grid,),
        in_specs=[
            pl.BlockSpec((blk, NGATE), lambda i: (i, 0)),
            pl.BlockSpec((_NC, blk, d), lambda i: (0, i, 0)),
            pl.BlockSpec((blk, d), lambda i: (i, 0)),
            pl.BlockSpec((d, NGATE), lambda i: (0, 0)),
        ],
        out_specs=[
            pl.BlockSpec((blk, d), lambda i: (i, 0)),
            pl.BlockSpec((blk, d), lambda i: (i, 0)),
        ],
        out_shape=[
            jax.ShapeDtypeStruct((n, d), jnp.float32),
            jax.ShapeDtypeStruct((n, d), jnp.float32),
        ],
    )

    fc = pl.pallas_call(
        _fc_body,
        grid=(grid,),
        in_specs=[
            pl.BlockSpec((_NC, blk, d), lambda i: (0, i, 0)),
            pl.BlockSpec((d, fc_W.shape[0]), lambda i: (0, 0)),
            pl.BlockSpec((1, fc_W.shape[0]), lambda i: (0, 0)),
        ],
        out_specs=pl.BlockSpec((blk, fc_W.shape[0]), lambda i: (i, 0)),
        out_shape=jax.ShapeDtypeStruct((n, fc_W.shape[0]), jnp.float32),
    )

    xw, h, c = k1(x, W_ihT, bi, bh)
    for _ in range(9):
        p = spmm(h, src, dst, w)
        h, c = cell(xw, p, c, W_hhT)
    p = spmm(h, src, dst, w)
    return fc(p, fc_WT, fb)


# speedup vs baseline: 3.5097x; 1.0002x over previous
import functools

import jax
import jax.numpy as jnp
from jax import lax
from jax.experimental import pallas as pl
from jax.experimental.pallas import tpu as pltpu
from jax.experimental.pallas import tpu_sc as plsc

NHID = 128
NGATE = 4 * NHID

_NC = 2
_NS = 16
_L = 16
_NW = _NC * _NS
_CHUNK = 128


def _splat(wv, k):
    dnums = lax.GatherDimensionNumbers(
        offset_dims=(), collapsed_slice_dims=(0,), start_index_map=(0,))
    idx = jnp.full((_L, 1), k, dtype=jnp.int32)
    return lax.gather(wv, idx, dnums, slice_sizes=(1,),
                      mode=lax.GatherScatterMode.PROMISE_IN_BOUNDS)


def _make_spmm(n, e_pad, d):
    cpw = e_pad // (_NW * _CHUNK)
    half = cpw // 2
    stripe = (n // _NS) // 8 * 8
    tail = n - (_NS - 1) * stripe
    mesh = plsc.VectorSubcoreMesh(core_axis_name="c", subcore_axis_name="s")

    @functools.partial(
        pl.kernel,
        out_type=jax.ShapeDtypeStruct((_NC, n, d), jnp.float32),
        mesh=mesh,
        scratch_types=[
            pltpu.VMEM((6, _CHUNK), jnp.int32),
            pltpu.VMEM((2, _CHUNK, d), jnp.float32),
            pltpu.VMEM_SHARED((n, d), jnp.float32),
            pltpu.SemaphoreType.DMA,
            pltpu.SemaphoreType.DMA,
            pltpu.SemaphoreType.DMA,
            pltpu.SemaphoreType.DMA,
        ],
    )
    def spmm(h_hbm, ids_hbm, out_hbm, id_v, rows_v, acc,
             si0, si1, sr0, sr1):
        cid = lax.axis_index("c")
        sid = lax.axis_index("s")
        wid = sid * _NC + cid
        sem_i = (si0, si1)
        sem_r = (sr0, sr1)

        zeros = jnp.zeros((_L,), jnp.float32)

        def zrow(i, _):
            for j in range(d // _L):
                rows_v[0, i, pl.ds(j * _L, _L)] = zeros
            return 0

        lax.fori_loop(0, _CHUNK, zrow, 0)
        r0 = sid * stripe
        nfull = stripe // _CHUNK
        rem = stripe - nfull * _CHUNK

        def crow(b, _):
            pltpu.sync_copy(rows_v.at[0],
                            acc.at[pl.ds(r0 + b * _CHUNK, _CHUNK)])
            return 0

        lax.fori_loop(0, nfull, crow, 0)
        if rem:
            pltpu.sync_copy(rows_v.at[0, pl.ds(0, rem)],
                            acc.at[pl.ds(r0 + nfull * _CHUNK, rem)])

        @pl.when(sid == _NS - 1)
        def _():
            pltpu.sync_copy(rows_v.at[0, pl.ds(0, tail - stripe)],
                            acc.at[pl.ds(r0 + stripe, tail - stripe)])

        plsc.subcore_barrier()

        c0 = wid * cpw

        def idx_copy(b, c):
            return pltpu.make_async_copy(ids_hbm.at[c],
                                         id_v.at[pl.ds(3 * b, 3)], sem_i[b])

        def row_gather(b):
            return pltpu.make_async_copy(h_hbm.at[id_v.at[3 * b]],
                                         rows_v.at[b], sem_r[b])

        def mul_scatter(b):
            def group_body(g2, _):
                wv = lax.bitcast_convert_type(
                    id_v[3 * b + 2, pl.ds(g2 * _L, _L)], jnp.float32)
                for k in range(_L):
                    ws = _splat(wv, k)
                    ei = g2 * _L + k
                    for j in range(d // _L):
                        sl = pl.ds(j * _L, _L)
                        rows_v[b, ei, sl] = rows_v[b, ei, sl] * ws
                return 0

            lax.fori_loop(0, _CHUNK // _L, group_body, 0)
            pltpu.sync_copy(rows_v.at[b], acc.at[id_v.at[3 * b + 1]],
                            add=True)

        idx_copy(0, c0).start()
        idx_copy(1, c0 + 1).start()
        idx_copy(0, c0).wait()
        row_gather(0).start()

        def pipe_body(i, _):
            idx_copy(1, c0 + 2 * i + 1).wait()
            row_gather(1).start()
            row_gather(0).wait()
            mul_scatter(0)

            @pl.when(i < half - 1)
            def _():
                idx_copy(0, c0 + 2 * i + 2).start()

            @pl.when(i < half - 1)
            def _():
                idx_copy(0, c0 + 2 * i + 2).wait()
                row_gather(0).start()

            row_gather(1).wait()
            mul_scatter(1)

            @pl.when(i < half - 1)
            def _():
                idx_copy(1, c0 + 2 * i + 3).start()

            return 0

        lax.fori_loop(0, half, pipe_body, 0)
        plsc.subcore_barrier()
        pltpu.sync_copy(acc.at[pl.ds(r0, stripe)],
                        out_hbm.at[cid, pl.ds(r0, stripe)])

        @pl.when(sid == _NS - 1)
        def _():
            pltpu.sync_copy(acc.at[pl.ds(r0 + stripe, tail - stripe)],
                            out_hbm.at[cid, pl.ds(r0 + stripe, tail - stripe)])

    return spmm


def _k1_body(x_ref, wT_ref, bi_ref, bh_ref, xw_ref, h_ref, c_ref):
    xw = jnp.dot(x_ref[...], wT_ref[...], preferred_element_type=jnp.float32)
    xw = xw + bi_ref[...] + bh_ref[...]
    xw_ref[...] = xw
    i = jax.nn.sigmoid(xw[:, 0 * NHID:1 * NHID])
    g = jnp.tanh(xw[:, 2 * NHID:3 * NHID])
    o = jax.nn.sigmoid(xw[:, 3 * NHID:4 * NHID])
    c = i * g
    c_ref[...] = c
    h_ref[...] = o * jnp.tanh(c)


def _cell_body(xw_ref, p_ref, c_ref, whT_ref, h_ref, cn_ref):
    h_in = p_ref[0] + p_ref[1]
    gates = xw_ref[...] + jnp.dot(h_in, whT_ref[...],
                                  preferred_element_type=jnp.float32)
    i = jax.nn.sigmoid(gates[:, 0 * NHID:1 * NHID])
    f = jax.nn.sigmoid(gates[:, 1 * NHID:2 * NHID])
    g = jnp.tanh(gates[:, 2 * NHID:3 * NHID])
    o = jax.nn.sigmoid(gates[:, 3 * NHID:4 * NHID])
    c = f * c_ref[...] + i * g
    cn_ref[...] = c
    h_ref[...] = o * jnp.tanh(c)


def _fc_body(p_ref, fwT_ref, fb_ref, o_ref):
    h = jax.nn.relu(p_ref[0] + p_ref[1])
    o_ref[...] = jnp.dot(h, fwT_ref[...],
                         preferred_element_type=jnp.float32) + fb_ref[...]


def kernel(x, edge_index, edge_weight, W_ih, W_hh, b_ih, b_hh, fc_W, fc_b):
    n, n_in = x.shape
    e = edge_index.shape[1]
    d = W_hh.shape[1]

    W_ihT = W_ih.T
    W_hhT = W_hh.T
    fc_WT = fc_W.T
    bi = b_ih.reshape(1, NGATE)
    bh = b_hh.reshape(1, NGATE)
    fb = fc_b.reshape(1, fc_W.shape[0])

    per_blk = _NW * _CHUNK * 2
    e_pad = ((e + per_blk - 1) // per_blk) * per_blk
    pad = e_pad - e
    dst = jnp.pad(edge_index[0], (0, pad))
    src = jnp.pad(edge_index[1], (0, pad))
    w = jnp.pad(edge_weight, (0, pad))
    ids = jnp.stack([
        src.astype(jnp.int32).reshape(-1, _CHUNK),
        dst.astype(jnp.int32).reshape(-1, _CHUNK),
        lax.bitcast_convert_type(w, jnp.int32).reshape(-1, _CHUNK),
    ], axis=1)

    spmm = _make_spmm(n, e_pad, d)

    blk = 1000
    grid = n // blk

    k1 = pl.pallas_call(
        _k1_body,
        grid=(grid,),
        in_specs=[
            pl.BlockSpec((blk, n_in), lambda i: (i, 0)),
            pl.BlockSpec((n_in, NGATE), lambda i: (0, 0)),
            pl.BlockSpec((1, NGATE), lambda i: (0, 0)),
            pl.BlockSpec((1, NGATE), lambda i: (0, 0)),
        ],
        out_specs=[
            pl.BlockSpec((blk, NGATE), lambda i: (i, 0)),
            pl.BlockSpec((blk, d), lambda i: (i, 0)),
            pl.BlockSpec((blk, d), lambda i: (i, 0)),
        ],
        out_shape=[
            jax.ShapeDtypeStruct((n, NGATE), jnp.float32),
            jax.ShapeDtypeStruct((n, d), jnp.float32),
            jax.ShapeDtypeStruct((n, d), jnp.float32),
        ],
    )

    cell = pl.pallas_call(
        _cell_body,
        grid=(grid,),
        in_specs=[
            pl.BlockSpec((blk, NGATE), lambda i: (i, 0)),
            pl.BlockSpec((_NC, blk, d), lambda i: (0, i, 0)),
            pl.BlockSpec((blk, d), lambda i: (i, 0)),
            pl.BlockSpec((d, NGATE), lambda i: (0, 0)),
        ],
        out_specs=[
            pl.BlockSpec((blk, d), lambda i: (i, 0)),
            pl.BlockSpec((blk, d), lambda i: (i, 0)),
        ],
        out_shape=[
            jax.ShapeDtypeStruct((n, d), jnp.float32),
            jax.ShapeDtypeStruct((n, d), jnp.float32),
        ],
    )

    fc = pl.pallas_call(
        _fc_body,
        grid=(grid,),
        in_specs=[
            pl.BlockSpec((_NC, blk, d), lambda i: (0, i, 0)),
            pl.BlockSpec((d, fc_W.shape[0]), lambda i: (0, 0)),
            pl.BlockSpec((1, fc_W.shape[0]), lambda i: (0, 0)),
        ],
        out_specs=pl.BlockSpec((blk, fc_W.shape[0]), lambda i: (i, 0)),
        out_shape=jax.ShapeDtypeStruct((n, fc_W.shape[0]), jnp.float32),
    )

    xw, h, c = k1(x, W_ihT, bi, bh)
    for _ in range(9):
        p = spmm(h, ids)
        h, c = cell(xw, p, c, W_hhT)
    p = spmm(h, ids)
    return fc(p, fc_WT, fb)
